# Initial kernel scaffold; baseline (speedup 1.0000x reference)
import jax
import jax.numpy as jnp
from jax.experimental import pallas as pl


def kernel(x, edge_index, W, b, prelu_w):
    raise NotImplementedError("write your pallas kernel here")


# speedup vs baseline: 15.8823x; 15.8823x over previous
import functools

import jax
import jax.numpy as jnp
from jax import lax
from jax.experimental import pallas as pl
from jax.experimental.pallas import tpu as pltpu
from jax.experimental.pallas import tpu_sc as plsc

N = 10000
E = 320000
D = 128

NC = 2
NS = 16
NW = NC * NS

CHUNK = 80
EPW = E // NW
STEPS = EPW // CHUNK
RPS = 624
TAIL0 = NS * RPS
TAILN = N - TAIL0

DEG_W = 128


def _vector_mesh():
    return plsc.VectorSubcoreMesh(core_axis_name="c", subcore_axis_name="s")


def _sc_degree(dst, zeros_deg, ones_chunk):
    @functools.partial(
        pl.kernel,
        mesh=_vector_mesh(),
        out_type=jax.ShapeDtypeStruct((NC, N, DEG_W), jnp.float32),
        scratch_types=[
            pltpu.VMEM((CHUNK,), jnp.int32),
            pltpu.VMEM((CHUNK, DEG_W), jnp.float32),
            pltpu.VMEM_SHARED((N, DEG_W), jnp.float32),
        ],
    )
    def k(dst_hbm, zeros_hbm, ones_hbm, out_hbm, didx_v, ones_v, deg_sp):
        c = lax.axis_index("c")
        s = lax.axis_index("s")
        stripe = pl.ds(s * jnp.int32(RPS), RPS)
        tail = pl.ds(TAIL0, TAILN)
        pltpu.sync_copy(zeros_hbm.at[stripe], deg_sp.at[stripe])

        @pl.when(s == NS - 1)
        def _():
            pltpu.sync_copy(zeros_hbm.at[tail], deg_sp.at[tail])

        pltpu.sync_copy(ones_hbm, ones_v)
        plsc.subcore_barrier()

        base = (c * jnp.int32(NS) + s) * jnp.int32(EPW)

        @pl.loop(jnp.int32(0), jnp.int32(STEPS))
        def _(i):
            off = base + jnp.int32(i) * jnp.int32(CHUNK)
            pltpu.sync_copy(dst_hbm.at[pl.ds(off, CHUNK)], didx_v)
            pltpu.sync_copy(ones_v, deg_sp.at[didx_v], add=True)

        plsc.subcore_barrier()
        pltpu.sync_copy(deg_sp.at[stripe], out_hbm.at[c].at[stripe])

        @pl.when(s == NS - 1)
        def _():
            pltpu.sync_copy(deg_sp.at[tail], out_hbm.at[c].at[tail])

    return k(dst, zeros_deg, ones_chunk)


def _sc_aggregate(h2, src, dst):
    @functools.partial(
        pl.kernel,
        mesh=_vector_mesh(),
        out_type=jax.ShapeDtypeStruct((NC, N, D), jnp.float32),
        scratch_types=[
            pltpu.VMEM((CHUNK,), jnp.int32),
            pltpu.VMEM((CHUNK,), jnp.int32),
            pltpu.VMEM((CHUNK, D), jnp.float32),
            pltpu.VMEM_SHARED((N, D), jnp.float32),
        ],
    )
    def k(h2_hbm, src_hbm, dst_hbm, out_hbm, sidx_v, didx_v, rows_v, agg_sp):
        c = lax.axis_index("c")
        s = lax.axis_index("s")
        stripe = pl.ds(s * jnp.int32(RPS), RPS)
        tail = pl.ds(TAIL0, TAILN)
        pltpu.sync_copy(h2_hbm.at[stripe], agg_sp.at[stripe])

        @pl.when(s == NS - 1)
        def _():
            pltpu.sync_copy(h2_hbm.at[tail], agg_sp.at[tail])

        plsc.subcore_barrier()

        base = (c * jnp.int32(NS) + s) * jnp.int32(EPW)

        @pl.loop(jnp.int32(0), jnp.int32(STEPS))
        def _(i):
            off = base + jnp.int32(i) * jnp.int32(CHUNK)
            pltpu.sync_copy(src_hbm.at[pl.ds(off, CHUNK)], sidx_v)
            pltpu.sync_copy(dst_hbm.at[pl.ds(off, CHUNK)], didx_v)
            pltpu.sync_copy(h2_hbm.at[sidx_v], rows_v)
            pltpu.sync_copy(rows_v, agg_sp.at[didx_v], add=True)

        plsc.subcore_barrier()
        pltpu.sync_copy(agg_sp.at[stripe], out_hbm.at[c].at[stripe])

        @pl.when(s == NS - 1)
        def _():
            pltpu.sync_copy(agg_sp.at[tail], out_hbm.at[c].at[tail])

    return k(h2, src, dst)


_BLK = 1000


def _tc_h2(x, W, degp):
    def body(x_ref, w_ref, d_ref, o_ref):
        deg = d_ref[0, :, 0] + d_ref[1, :, 0] + 1.0
        dinv = lax.rsqrt(deg)
        h = jnp.dot(x_ref[...], w_ref[...], preferred_element_type=jnp.float32)
        o_ref[...] = h * dinv[:, None]

    return pl.pallas_call(
        body,
        grid=(N // _BLK,),
        in_specs=[
            pl.BlockSpec((_BLK, D), lambda i: (i, jnp.int32(0))),
            pl.BlockSpec((D, D), lambda i: (jnp.int32(0), jnp.int32(0))),
            pl.BlockSpec((NC, _BLK, DEG_W), lambda i: (jnp.int32(0), i, jnp.int32(0))),
        ],
        out_specs=pl.BlockSpec((_BLK, D), lambda i: (i, jnp.int32(0))),
        out_shape=jax.ShapeDtypeStruct((N, D), jnp.float32),
    )(x, W, degp)


def _tc_finish(aggp, h2, degp, b2, pw2):
    def body(a_ref, h2_ref, d_ref, b_ref, p_ref, o_ref):
        deg = d_ref[0, :, 0] + d_ref[1, :, 0] + 1.0
        dinv = lax.rsqrt(deg)
        ssum = a_ref[0] + a_ref[1] - h2_ref[...]
        pre = ssum * dinv[:, None] + b_ref[...]
        o_ref[...] = jnp.where(pre > 0, pre, pre * p_ref[...])

    return pl.pallas_call(
        body,
        grid=(N // _BLK,),
        in_specs=[
            pl.BlockSpec((NC, _BLK, D), lambda i: (jnp.int32(0), i, jnp.int32(0))),
            pl.BlockSpec((_BLK, D), lambda i: (i, jnp.int32(0))),
            pl.BlockSpec((NC, _BLK, DEG_W), lambda i: (jnp.int32(0), i, jnp.int32(0))),
            pl.BlockSpec((1, D), lambda i: (jnp.int32(0), jnp.int32(0))),
            pl.BlockSpec((1, D), lambda i: (jnp.int32(0), jnp.int32(0))),
        ],
        out_specs=pl.BlockSpec((_BLK, D), lambda i: (i, jnp.int32(0))),
        out_shape=jax.ShapeDtypeStruct((N, D), jnp.float32),
    )(aggp, h2, degp, b2, pw2)


def kernel(x, edge_index, W, b, prelu_w):
    ei = edge_index.astype(jnp.int32)
    src = ei[0]
    dst = ei[1]
    x = x.astype(jnp.float32)
    W = W.astype(jnp.float32)

    zeros_deg = jnp.zeros((N, DEG_W), jnp.float32)
    ones_chunk = jnp.ones((CHUNK, DEG_W), jnp.float32)

    degp = _sc_degree(dst, zeros_deg, ones_chunk)
    h2 = _tc_h2(x, W, degp)
    aggp = _sc_aggregate(h2, src, dst)
    out = _tc_finish(aggp, h2, degp,
                     b.reshape(1, D).astype(jnp.float32),
                     prelu_w.reshape(1, D).astype(jnp.float32))
    return out


# speedup vs baseline: 37.4273x; 2.3566x over previous
import dataclasses
import functools

import jax
import jax.numpy as jnp
from jax import lax
from jax.experimental import pallas as pl
from jax.experimental.pallas import tpu as pltpu
from jax.experimental.pallas import tpu_sc as plsc

N = 10000
E = 320000
D = 128

NC = 2
NS = 16
NW = NC * NS

CHUNK = 80
EPW = E // NW
STEPS = EPW // CHUNK
RPS = 624
TAIL0 = NS * RPS
TAILN = N - TAIL0

NPAD = 10240
VEC = 16


def _vector_mesh():
    return plsc.VectorSubcoreMesh(core_axis_name="c", subcore_axis_name="s")


def _sc_compiler_params():
    cp = pltpu.CompilerParams()
    if "needs_layout_passes" in pltpu.CompilerParams.__dataclass_fields__:
        cp = dataclasses.replace(cp, needs_layout_passes=False)
    return cp


def _sc_degree(ei2):
    @functools.partial(
        pl.kernel,
        mesh=_vector_mesh(),
        compiler_params=_sc_compiler_params(),
        out_type=jax.ShapeDtypeStruct((NC, NS, NPAD), jnp.float32),
        scratch_types=[
            pltpu.VMEM((EPW,), jnp.int32),
            pltpu.VMEM((NPAD,), jnp.float32),
        ],
    )
    def k(ei_hbm, out_hbm, didx_a, hist_v):
        c = lax.axis_index("c")
        s = lax.axis_index("s")
        w = c * jnp.int32(NS) + s
        pltpu.sync_copy(ei_hbm.at[jnp.int32(1), w, :], didx_a)

        zeros16 = jnp.zeros((VEC,), jnp.float32)
        ones16 = jnp.ones((VEC,), jnp.float32)

        @pl.loop(jnp.int32(0), jnp.int32(NPAD), step=jnp.int32(VEC))
        def _(i):
            hist_v[pl.ds(i, VEC)] = zeros16

        @pl.loop(jnp.int32(0), jnp.int32(EPW), step=jnp.int32(VEC))
        def _(i):
            idx = didx_a[pl.ds(i, VEC)]
            plsc.addupdate_scatter(hist_v, [idx], ones16)

        pltpu.sync_copy(hist_v, out_hbm.at[c, s, :])

    return k(ei2)


def _sc_aggregate(h2, ei2, ei3):
    @functools.partial(
        pl.kernel,
        mesh=_vector_mesh(),
        out_type=jax.ShapeDtypeStruct((NC, N, D), jnp.float32),
        scratch_types=[
            pltpu.VMEM((EPW,), jnp.int32),
            pltpu.VMEM((STEPS, CHUNK), jnp.int32),
            pltpu.VMEM((CHUNK, D), jnp.float32),
            pltpu.VMEM((CHUNK, D), jnp.float32),
            pltpu.VMEM_SHARED((N, D), jnp.float32),
            pltpu.SemaphoreType.DMA,
            pltpu.SemaphoreType.DMA,
        ],
    )
    def k(h2_hbm, ei2_hbm, ei3_hbm, out_hbm, sidx_a, didx_a, rows0, rows1,
          agg_sp, semg0, semg1):
        c = lax.axis_index("c")
        s = lax.axis_index("s")
        w = c * jnp.int32(NS) + s
        stripe = pl.ds(s * jnp.int32(RPS), RPS)
        tail = pl.ds(TAIL0, TAILN)

        pltpu.sync_copy(ei2_hbm.at[jnp.int32(0), w, :], sidx_a)
        pltpu.sync_copy(ei3_hbm.at[jnp.int32(1), w, :, :], didx_a)
        pltpu.sync_copy(h2_hbm.at[stripe], agg_sp.at[stripe])

        @pl.when(s == NS - 1)
        def _():
            pltpu.sync_copy(h2_hbm.at[tail], agg_sp.at[tail])

        plsc.subcore_barrier()

        def start_gather(j, rows, sem):
            pltpu.async_copy(
                h2_hbm.at[sidx_a.at[pl.ds(j * jnp.int32(CHUNK), CHUNK)]],
                rows, sem)

        def wait_gather(j, rows, sem):
            pltpu.make_async_copy(
                h2_hbm.at[sidx_a.at[pl.ds(j * jnp.int32(CHUNK), CHUNK)]],
                rows, sem).wait()

        def scatter(j, rows):
            pltpu.sync_copy(rows, agg_sp.at[didx_a.at[j]], add=True)

        start_gather(jnp.int32(0), rows0, semg0)

        @pl.loop(jnp.int32(0), jnp.int32((STEPS - 1) // 2))
        def _(k2):
            j = jnp.int32(2) * jnp.int32(k2)
            wait_gather(j, rows0, semg0)
            start_gather(j + 1, rows1, semg1)
            scatter(j, rows0)
            wait_gather(j + 1, rows1, semg1)
            start_gather(j + 2, rows0, semg0)
            scatter(j + 1, rows1)

        last = jnp.int32(STEPS - 1)
        wait_gather(last, rows0, semg0)
        scatter(last, rows0)

        plsc.subcore_barrier()
        pltpu.sync_copy(agg_sp.at[stripe], out_hbm.at[c].at[stripe])

        @pl.when(s == NS - 1)
        def _():
            pltpu.sync_copy(agg_sp.at[tail], out_hbm.at[c].at[tail])

    return k(h2, ei2, ei3)


_BLK = 1024
_GRID = NPAD // _BLK


def _tc_h2(x, W, degp):
    def body(x_ref, w_ref, d_ref, o_ref):
        deg = jnp.sum(d_ref[...], axis=0) + 1.0
        dinv = lax.rsqrt(deg)
        h = jnp.dot(x_ref[...], w_ref[...], preferred_element_type=jnp.float32)
        o_ref[...] = h * dinv[None, :].reshape(_BLK, 1)

    return pl.pallas_call(
        body,
        grid=(_GRID,),
        in_specs=[
            pl.BlockSpec((_BLK, D), lambda i: (i, jnp.int32(0))),
            pl.BlockSpec((D, D), lambda i: (jnp.int32(0), jnp.int32(0))),
            pl.BlockSpec((NW, _BLK), lambda i: (jnp.int32(0), i)),
        ],
        out_specs=pl.BlockSpec((_BLK, D), lambda i: (i, jnp.int32(0))),
        out_shape=jax.ShapeDtypeStruct((N, D), jnp.float32),
    )(x, W, degp)


def _tc_finish(aggp, h2, degp, b2, pw2):
    def body(a_ref, h2_ref, d_ref, b_ref, p_ref, o_ref):
        deg = jnp.sum(d_ref[...], axis=0) + 1.0
        dinv = lax.rsqrt(deg)
        ssum = a_ref[0] + a_ref[1] - h2_ref[...]
        pre = ssum * dinv[None, :].reshape(_BLK, 1) + b_ref[...]
        o_ref[...] = jnp.where(pre > 0, pre, pre * p_ref[...])

    return pl.pallas_call(
        body,
        grid=(_GRID,),
        in_specs=[
            pl.BlockSpec((NC, _BLK, D), lambda i: (jnp.int32(0), i, jnp.int32(0))),
            pl.BlockSpec((_BLK, D), lambda i: (i, jnp.int32(0))),
            pl.BlockSpec((NW, _BLK), lambda i: (jnp.int32(0), i)),
            pl.BlockSpec((1, D), lambda i: (jnp.int32(0), jnp.int32(0))),
            pl.BlockSpec((1, D), lambda i: (jnp.int32(0), jnp.int32(0))),
        ],
        out_specs=pl.BlockSpec((_BLK, D), lambda i: (i, jnp.int32(0))),
        out_shape=jax.ShapeDtypeStruct((N, D), jnp.float32),
    )(aggp, h2, degp, b2, pw2)


def kernel(x, edge_index, W, b, prelu_w):
    ei = edge_index.astype(jnp.int32)
    ei2 = ei.reshape(2, NW, EPW)
    ei3 = ei.reshape(2, NW, STEPS, CHUNK)
    x = x.astype(jnp.float32)
    W = W.astype(jnp.float32)

    degp = _sc_degree(ei2).reshape(NW, NPAD)
    h2 = _tc_h2(x, W, degp)
    aggp = _sc_aggregate(h2, ei2, ei3)
    out = _tc_finish(aggp, h2, degp,
                     b.reshape(1, D).astype(jnp.float32),
                     prelu_w.reshape(1, D).astype(jnp.float32))
    return out
